# baseline (speedup 1.0000x reference)
import jax
import jax.numpy as jnp
from jax.experimental import pallas as pl


def _ew_kernel(x_ref, o_ref):
    o_ref[...] = x_ref[...] * 2.0 - 1.0


def kernel(x):
    M, N = x.shape
    BM = 1024
    grid = (M // BM,)
    return pl.pallas_call(
        _ew_kernel,
        grid=grid,
        in_specs=[pl.BlockSpec((BM, N), lambda i: (i, 0))],
        out_specs=pl.BlockSpec((BM, N), lambda i: (i, 0)),
        out_shape=jax.ShapeDtypeStruct((M, N), x.dtype),
    )(x)
